# Initial kernel scaffold; baseline (speedup 1.0000x reference)
import jax
import jax.numpy as jnp
from jax.experimental import pallas as pl


def kernel(x, edge_index, edge_attr, params):
    raise NotImplementedError("write your pallas kernel here")


# speedup vs baseline: 1.0135x; 1.0135x over previous
import functools

import jax
import jax.numpy as jnp
import numpy as np
from jax.experimental import pallas as pl
from jax.experimental.pallas import tpu as pltpu

N = 1024
E = 4096
HID = 64
HEADS = 4
HD = HID // HEADS
LPE_K = 4
K_KEEP = int(np.ceil(0.6 * N))
KP = 640
EPS = 1e-5

_INV_SQRT2 = 0.7071067811865476


def _gelu(x):
    return x * 0.5 * (1.0 + jax.lax.erf(x * _INV_SQRT2))


def _prep_body(zp_ref, e1w_ref, e1b_ref, c1w_ref, c1b_ref, c2w_ref, c2b_ref,
               za_ref, zb_ref, coords_ref):
    zp = zp_ref[...]
    e1w = e1w_ref[...]
    za_ref[...] = jnp.dot(zp, e1w[:HID, :], preferred_element_type=jnp.float32)
    zb_ref[...] = (jnp.dot(zp, e1w[HID:, :], preferred_element_type=jnp.float32)
                   + e1b_ref[...])
    h = _gelu(jnp.dot(zp, c1w_ref[...], preferred_element_type=jnp.float32)
              + c1b_ref[...])
    coords_ref[...] = (jnp.dot(h, c2w_ref[...], preferred_element_type=jnp.float32)
                       + c2b_ref[...])


def _decoder_prep(zp, e1_W, e1_b, c1_W, c1_b, c2_W, c2_b):
    return pl.pallas_call(
        _prep_body,
        out_shape=(
            jax.ShapeDtypeStruct((KP, HID), jnp.float32),
            jax.ShapeDtypeStruct((KP, HID), jnp.float32),
            jax.ShapeDtypeStruct((KP, 8), jnp.float32),
        ),
    )(zp, e1_W, e1_b.reshape(1, HID), c1_W, c1_b.reshape(1, 32),
      jnp.pad(c2_W, ((0, 0), (0, 6))),
      jnp.pad(c2_b.reshape(1, 2), ((0, 0), (0, 6))))


_SI = 16
_BI = 128
_BJ = 128


def _adj_body(za_ref, zb_ref, w2_ref, b2_ref, w3_ref, b3_ref, out_ref):
    zb = zb_ref[...]
    w2 = w2_ref[...]
    b2 = b2_ref[...]
    w3 = w3_ref[...]
    b3 = b3_ref[0, 0]

    def chunk(p, _):
        zi = za_ref[pl.ds(p * _SI, _SI), :]
        h1 = _gelu(zi[:, None, :] + zb[None, :, :])
        h1f = h1.reshape(_SI * _BJ, HID)
        h2 = _gelu(jnp.dot(h1f, w2, preferred_element_type=jnp.float32) + b2)
        t = jnp.sum(h2 * w3, axis=1) + b3
        out_ref[pl.ds(p * _SI, _SI), :] = t.reshape(_SI, _BJ)
        return 0

    jax.lax.fori_loop(0, _BI // _SI, chunk, 0)


def _decoder_adj(za, zb, e2_W, e2_b, e3_W, e3_b):
    grid = (KP // _BI, KP // _BJ)
    return pl.pallas_call(
        _adj_body,
        grid=grid,
        in_specs=[
            pl.BlockSpec((_BI, HID), lambda i, j: (i, 0)),
            pl.BlockSpec((_BJ, HID), lambda i, j: (j, 0)),
            pl.BlockSpec((HID, 32), lambda i, j: (0, 0)),
            pl.BlockSpec((1, 32), lambda i, j: (0, 0)),
            pl.BlockSpec((1, 32), lambda i, j: (0, 0)),
            pl.BlockSpec((1, 1), lambda i, j: (0, 0)),
        ],
        out_specs=pl.BlockSpec((_BI, _BJ), lambda i, j: (i, j)),
        out_shape=jax.ShapeDtypeStruct((KP, KP), jnp.float32),
    )(za, zb, e2_W, e2_b.reshape(1, 32), e3_W.reshape(1, 32),
      e3_b.reshape(1, 1))


def kernel(x, edge_index, edge_attr, params):
    p = params
    n = x.shape[0]
    row = edge_index[0]
    col = edge_index[1]

    deg = jax.ops.segment_sum(jnp.ones_like(row, dtype=jnp.float32), row,
                              num_segments=n)
    dis = jnp.where(deg > 0, deg ** -0.5, 0.0)
    w_off = -dis[row] * dis[col]
    L = jnp.eye(n, dtype=jnp.float32)
    L = L.at[row, col].add(jnp.where(row == col, 0.0, w_off))
    _, eigvecs = jnp.linalg.eigh(L)
    pe = jax.lax.stop_gradient(eigvecs[:, 1:LPE_K + 1])

    xx = x + pe @ p['lpe_W'] + p['lpe_b']
    xx = xx @ p['proj_W'] + p['proj_b']
    scale = 1.0 / np.sqrt(HD)
    for lp in p['layers']:
        q = (xx @ lp['Wq'] + lp['bq']).reshape(n, HEADS, HD)
        k = (xx @ lp['Wk'] + lp['bk']).reshape(n, HEADS, HD)
        v = (xx @ lp['Wv'] + lp['bv']).reshape(n, HEADS, HD)
        e = (edge_attr @ lp['We']).reshape(-1, HEADS, HD)
        k_j = k[row] + e
        v_j = v[row] + e
        alpha = (q[col] * k_j).sum(-1) * scale
        amax = jax.ops.segment_max(alpha, col, num_segments=n)
        ex = jnp.exp(alpha - amax[col])
        den = jax.ops.segment_sum(ex, col, num_segments=n)
        attn = ex / (den[col] + 1e-16)
        out = jax.ops.segment_sum(v_j * attn[:, :, None], col,
                                  num_segments=n).reshape(n, HID)
        x_r = xx @ lp['Ws'] + lp['bs']
        beta = jax.nn.sigmoid(
            jnp.concatenate([out, x_r, out - x_r], axis=-1) @ lp['Wb'])
        out = beta * x_r + (1.0 - beta) * out
        h = xx + out
        mu = h.mean(axis=-1, keepdims=True)
        var = jnp.mean((h - mu) ** 2, axis=-1, keepdims=True)
        h = (h - mu) / jnp.sqrt(var + EPS) * lp['ln_g'] + lp['ln_b']
        xx = jax.nn.gelu(h, approximate=False)

    w = p['pool_w']
    score = jnp.tanh((xx @ w) / jnp.linalg.norm(w))
    vals, perm = jax.lax.top_k(score, K_KEEP)
    z_s = xx[perm] * vals[:, None]

    zp = jnp.pad(z_s, ((0, KP - K_KEEP), (0, 0)))
    za, zb, coords_p = _decoder_prep(zp, p['e1_W'], p['e1_b'], p['c1_W'],
                                     p['c1_b'], p['c2_W'], p['c2_b'])
    adj_p = _decoder_adj(za, zb, p['e2_W'], p['e2_b'], p['e3_W'], p['e3_b'])
    adj = adj_p[:K_KEEP, :K_KEEP]
    adj = (adj + adj.T) / 2.0
    coords = coords_p[:K_KEEP, :2]
    batch_out = jnp.zeros((K_KEEP,), dtype=jnp.int32)
    return coords, adj, batch_out, perm
